# baseline (speedup 1.0000x reference)
import jax
import jax.numpy as jnp
import numpy as np
from jax.experimental import pallas as pl
from jax.experimental.pallas import tpu as pltpu

_BM = 400
_CB = 2048


def _pass_a_kernel(x_ref, adj_ref, w1_ref, b1_ref, w2_ref,
                   h2_out, op_out, s1_ref, h2v_ref, x_s, x_sem):
    i = pl.program_id(0)
    nb = pl.num_programs(0)

    @pl.when(i == 0)
    def _():
        xcp = pltpu.make_async_copy(x_ref, x_s, x_sem)
        xcp.start()
        h2v_ref[...] = jnp.zeros_like(h2v_ref)
        xcp.wait()
        s1_ref[...] = jnp.dot(x_s[...], w1_ref[...],
                              preferred_element_type=jnp.float32)

    t = nb - 1 - i
    a = adj_ref[...]
    acc = jnp.dot(a, s1_ref[...], preferred_element_type=jnp.float32)
    hb = jnp.maximum(acc + b1_ref[...], 0.0)
    h2blk = jnp.dot(hb, w2_ref[...], preferred_element_type=jnp.float32)
    h2v_ref[pl.ds(t * _BM, _BM), :] = h2blk
    h2_out[...] = h2blk
    op_out[...] = jnp.dot(a, h2v_ref[...],
                          preferred_element_type=jnp.float32)


_RAG = 10000 - 4 * _CB


def _pass_b_kernel(tb_ref, cb_ref, mz_ref, fst_ref, lst_ref, rg_ref,
                   adj_ref, h2_ref, op_ref, b2_ref, out_ref):
    i = pl.program_id(0)
    mz = mz_ref[i]
    c = cb_ref[i]
    t = tb_ref[i]
    fst = fst_ref[i]

    h2s = h2_ref[pl.ds(c * _CB, _CB), :]
    rowid = jax.lax.broadcasted_iota(jnp.int32, h2s.shape, 0)
    h2m = jnp.where(rowid < mz, h2s, 0.0)

    def emit(a_op):
        ctb = jnp.dot(a_op, h2m, preferred_element_type=jnp.float32)

        @pl.when(fst == 1)
        def _():
            out_ref[...] = op_ref[pl.ds(t * _BM, _BM), :] + ctb

        @pl.when(fst == 0)
        def _():
            out_ref[...] = out_ref[...] + ctb

    @pl.when(rg_ref[i] == 0)
    def _():
        emit(adj_ref[...])

    @pl.when(rg_ref[i] == 1)
    def _():
        colid = jax.lax.broadcasted_iota(jnp.int32, adj_ref.shape, 1)
        emit(jnp.where(colid < _RAG, adj_ref[...], 0.0))

    @pl.when(lst_ref[i] == 1)
    def _():
        o = out_ref[...] + b2_ref[...]
        mx = jnp.max(o, axis=1, keepdims=True)
        shifted = o - mx
        lse = jnp.log(jnp.sum(jnp.exp(shifted), axis=1, keepdims=True))
        out_ref[...] = shifted - lse


def _pass_b_schedule(n):
    nb = n // _BM
    ncb = -(-n // _CB)
    tb, cb, mz, fst, lst, rg = [], [], [], [], [], []
    for t in range(nb):
        end = t * _BM
        kend = -(-end // _CB)
        if kend == 0:
            tb.append(t); cb.append(0); mz.append(0)
            fst.append(1); lst.append(1); rg.append(0)
            continue
        for c in range(kend):
            tb.append(t)
            cb.append(c)
            mz.append(_CB if c < kend - 1 else end - (kend - 1) * _CB)
            fst.append(1 if c == 0 else 0)
            lst.append(1 if c == kend - 1 else 0)
            rg.append(1 if c == ncb - 1 else 0)
    arrs = (tb, cb, mz, fst, lst, rg)
    return tuple(jnp.asarray(np.array(a, dtype=np.int32)) for a in arrs)


@jax.jit
def kernel(x, adj, W1, b1, W2, b2):
    n, nfeat = x.shape
    nhid = W1.shape[1]
    nclass = W2.shape[1]
    b1r = b1.reshape(1, nhid)
    b2r = b2.reshape(1, nclass)
    nb = n // _BM
    ncb = -(-n // _CB)

    h2, opart = pl.pallas_call(
        _pass_a_kernel,
        grid=(nb,),
        in_specs=[
            pl.BlockSpec(memory_space=pl.ANY),
            pl.BlockSpec((_BM, n), lambda i: (nb - 1 - i, 0)),
            pl.BlockSpec((nfeat, nhid), lambda i: (0, 0)),
            pl.BlockSpec((1, nhid), lambda i: (0, 0)),
            pl.BlockSpec((nhid, nclass), lambda i: (0, 0)),
        ],
        out_specs=[
            pl.BlockSpec((_BM, nclass), lambda i: (nb - 1 - i, 0)),
            pl.BlockSpec((_BM, nclass), lambda i: (nb - 1 - i, 0)),
        ],
        out_shape=[
            jax.ShapeDtypeStruct((n, nclass), jnp.float32),
            jax.ShapeDtypeStruct((n, nclass), jnp.float32),
        ],
        scratch_shapes=[
            pltpu.VMEM((n, nhid), jnp.float32),
            pltpu.VMEM((n, nclass), jnp.float32),
            pltpu.VMEM((n, nfeat), jnp.float32),
            pltpu.SemaphoreType.DMA,
        ],
        compiler_params=pltpu.CompilerParams(
            dimension_semantics=("arbitrary",),
            vmem_limit_bytes=58 * 1024 * 1024),
    )(x, adj, W1, b1r, W2)

    h2p = jnp.zeros((ncb * _CB, nclass), jnp.float32).at[:n].set(h2)

    tb, cb, mz, fst, lst, rg = _pass_b_schedule(n)
    nsteps = tb.shape[0]

    grid_spec = pltpu.PrefetchScalarGridSpec(
        num_scalar_prefetch=6,
        grid=(nsteps,),
        in_specs=[
            pl.BlockSpec((_BM, _CB),
                         lambda i, tb, cb, *_: (tb[i], cb[i])),
            pl.BlockSpec((ncb * _CB, nclass), lambda i, *_: (0, 0)),
            pl.BlockSpec((n, nclass), lambda i, *_: (0, 0)),
            pl.BlockSpec((1, nclass), lambda i, *_: (0, 0)),
        ],
        out_specs=pl.BlockSpec((_BM, nclass),
                               lambda i, tb, *_: (tb[i], 0)),
    )

    return pl.pallas_call(
        _pass_b_kernel,
        grid_spec=grid_spec,
        out_shape=jax.ShapeDtypeStruct((n, nclass), jnp.float32),
        compiler_params=pltpu.CompilerParams(
            dimension_semantics=("arbitrary",),
            vmem_limit_bytes=58 * 1024 * 1024),
    )(tb, cb, mz, fst, lst, rg, adj, h2p, opart, b2r)


# speedup vs baseline: 1.3661x; 1.3661x over previous
import jax
import jax.numpy as jnp
import numpy as np
from jax.experimental import pallas as pl
from jax.experimental.pallas import tpu as pltpu

_BM = 400
_CB = 2048


def _pass_a_kernel(x_ref, adj_ref, w1_ref, b1_ref, w2_ref,
                   h2_out, op_out, sh_ref, x_s, x_sem):
    i = pl.program_id(0)
    nb = pl.num_programs(0)
    nhid = w1_ref.shape[1]
    nclass = w2_ref.shape[1]

    @pl.when(i == 0)
    def _():
        xcp = pltpu.make_async_copy(x_ref, x_s, x_sem)
        xcp.start()
        sh_ref[...] = jnp.zeros_like(sh_ref)
        xcp.wait()
        sh_ref[:, :nhid] = jnp.dot(x_s[...], w1_ref[...],
                                   preferred_element_type=jnp.float32)

    t = nb - 1 - i
    a = adj_ref[...]
    big = jnp.dot(a, sh_ref[...], preferred_element_type=jnp.float32)
    hb = jnp.maximum(big[:, :nhid] + b1_ref[...], 0.0)
    h2blk = jnp.dot(hb, w2_ref[...], preferred_element_type=jnp.float32)
    sh_ref[pl.ds(t * _BM, _BM), nhid:nhid + nclass] = h2blk
    h2_out[...] = h2blk
    op_out[...] = big[:, nhid:nhid + nclass]


_RAG = 10000 - 4 * _CB


def _pass_b_kernel(tb_ref, cb_ref, mz_ref, fst_ref, lst_ref, rg_ref,
                   adj_ref, h2_ref, op_ref, b2_ref, out_ref):
    i = pl.program_id(0)
    mz = mz_ref[i]
    c = cb_ref[i]
    t = tb_ref[i]
    fst = fst_ref[i]

    h2s = h2_ref[pl.ds(c * _CB, _CB), :]
    rowid = jax.lax.broadcasted_iota(jnp.int32, h2s.shape, 0)
    h2m = jnp.where(rowid < mz, h2s, 0.0)

    def emit(a_op):
        ctb = jnp.dot(a_op, h2m, preferred_element_type=jnp.float32)

        @pl.when(fst == 1)
        def _():
            out_ref[...] = op_ref[pl.ds(t * _BM, _BM), :] + ctb

        @pl.when(fst == 0)
        def _():
            out_ref[...] = out_ref[...] + ctb

    @pl.when(rg_ref[i] == 0)
    def _():
        emit(adj_ref[...])

    @pl.when(rg_ref[i] == 1)
    def _():
        colid = jax.lax.broadcasted_iota(jnp.int32, adj_ref.shape, 1)
        emit(jnp.where(colid < _RAG, adj_ref[...], 0.0))

    @pl.when(lst_ref[i] == 1)
    def _():
        o = out_ref[...] + b2_ref[...]
        mx = jnp.max(o, axis=1, keepdims=True)
        shifted = o - mx
        lse = jnp.log(jnp.sum(jnp.exp(shifted), axis=1, keepdims=True))
        out_ref[...] = shifted - lse


def _pass_b_schedule(n):
    nb = n // _BM
    ncb = -(-n // _CB)
    tb, cb, mz, fst, lst, rg = [], [], [], [], [], []
    for t in range(nb):
        end = (t + 1) * _BM
        kend = -(-end // _CB)
        for c in range(kend):
            tb.append(t)
            cb.append(c)
            mz.append(_CB if c < kend - 1 else end - (kend - 1) * _CB)
            fst.append(1 if c == 0 else 0)
            lst.append(1 if c == kend - 1 else 0)
            rg.append(1 if c == ncb - 1 else 0)
    arrs = (tb, cb, mz, fst, lst, rg)
    return tuple(jnp.asarray(np.array(a, dtype=np.int32)) for a in arrs)


@jax.jit
def kernel(x, adj, W1, b1, W2, b2):
    n, nfeat = x.shape
    nhid = W1.shape[1]
    nclass = W2.shape[1]
    b1r = b1.reshape(1, nhid)
    b2r = b2.reshape(1, nclass)
    nb = n // _BM
    ncb = -(-n // _CB)

    h2, opart = pl.pallas_call(
        _pass_a_kernel,
        grid=(nb,),
        in_specs=[
            pl.BlockSpec(memory_space=pl.ANY),
            pl.BlockSpec((_BM, n), lambda i: (nb - 1 - i, 0)),
            pl.BlockSpec((nfeat, nhid), lambda i: (0, 0)),
            pl.BlockSpec((1, nhid), lambda i: (0, 0)),
            pl.BlockSpec((nhid, nclass), lambda i: (0, 0)),
        ],
        out_specs=[
            pl.BlockSpec((_BM, nclass), lambda i: (nb - 1 - i, 0)),
            pl.BlockSpec((_BM, nclass), lambda i: (nb - 1 - i, 0)),
        ],
        out_shape=[
            jax.ShapeDtypeStruct((n, nclass), jnp.float32),
            jax.ShapeDtypeStruct((n, nclass), jnp.float32),
        ],
        scratch_shapes=[
            pltpu.VMEM((n, nhid + nclass), jnp.float32),
            pltpu.VMEM((n, nfeat), jnp.float32),
            pltpu.SemaphoreType.DMA,
        ],
        compiler_params=pltpu.CompilerParams(
            dimension_semantics=("arbitrary",),
            vmem_limit_bytes=58 * 1024 * 1024),
    )(x, adj, W1, b1r, W2)

    h2p = jnp.zeros((ncb * _CB, nclass), jnp.float32).at[:n].set(h2)

    tb, cb, mz, fst, lst, rg = _pass_b_schedule(n)
    nsteps = tb.shape[0]

    grid_spec = pltpu.PrefetchScalarGridSpec(
        num_scalar_prefetch=6,
        grid=(nsteps,),
        in_specs=[
            pl.BlockSpec((_BM, _CB),
                         lambda i, tb, cb, *_: (tb[i], cb[i])),
            pl.BlockSpec((ncb * _CB, nclass), lambda i, *_: (0, 0)),
            pl.BlockSpec((n, nclass), lambda i, *_: (0, 0)),
            pl.BlockSpec((1, nclass), lambda i, *_: (0, 0)),
        ],
        out_specs=pl.BlockSpec((_BM, nclass),
                               lambda i, tb, *_: (tb[i], 0)),
    )

    return pl.pallas_call(
        _pass_b_kernel,
        grid_spec=grid_spec,
        out_shape=jax.ShapeDtypeStruct((n, nclass), jnp.float32),
        compiler_params=pltpu.CompilerParams(
            dimension_semantics=("arbitrary",),
            vmem_limit_bytes=58 * 1024 * 1024),
    )(tb, cb, mz, fst, lst, rg, adj, h2p, opart, b2r)


# speedup vs baseline: 1.5799x; 1.1566x over previous
import jax
import jax.numpy as jnp
import numpy as np
from jax.experimental import pallas as pl
from jax.experimental.pallas import tpu as pltpu

_BM = 400
_GB = 5
_BMB = _BM * _GB
_CB = 2048


def _pass_a_kernel(x_ref, adj_ref, w1_ref, b1_ref, w2_ref,
                   h2_out, op_out, sh_ref, hstage, x_s, x_sem):
    i = pl.program_id(0)
    nb = pl.num_programs(0)
    nhid = w1_ref.shape[1]
    nclass = w2_ref.shape[1]

    @pl.when(i == 0)
    def _():
        xcp = pltpu.make_async_copy(x_ref, x_s, x_sem)
        xcp.start()
        sh_ref[...] = jnp.zeros_like(sh_ref)
        xcp.wait()
        sh_ref[:, :nhid] = jnp.dot(x_s[...], w1_ref[...],
                                   preferred_element_type=jnp.float32)

    t = nb - 1 - i

    @pl.when(((t + 1) % _GB == 0) & (i > 0))
    def _():
        sh_ref[pl.ds((t + 1) * _BM, _BMB), nhid:nhid + nclass] = (
            hstage[...])

    a = adj_ref[...]
    big = jnp.dot(a, sh_ref[...], preferred_element_type=jnp.float32)
    hb = jnp.maximum(big[:, :nhid] + b1_ref[...], 0.0)
    h2blk = jnp.dot(hb, w2_ref[...], preferred_element_type=jnp.float32)
    hstage[pl.ds((t % _GB) * _BM, _BM), :] = h2blk
    h2_out[...] = h2blk
    op_out[...] = big[:, nhid:nhid + nclass]


_RAG = 10000 - 4 * _CB


def _pass_b_kernel(tb_ref, cb_ref, mz_ref, fst_ref, lst_ref, rg_ref,
                   adj_ref, h2_ref, op_ref, b2_ref, out_ref):
    i = pl.program_id(0)
    mz = mz_ref[i]
    c = cb_ref[i]
    t = tb_ref[i]
    fst = fst_ref[i]

    h2s = h2_ref[pl.ds(c * _CB, _CB), :]
    rowid = jax.lax.broadcasted_iota(jnp.int32, h2s.shape, 0)
    h2m = jnp.where(rowid < mz, h2s, 0.0)

    def emit(a_op):
        ctb = jnp.dot(a_op, h2m, preferred_element_type=jnp.float32)

        @pl.when(fst == 1)
        def _():
            out_ref[...] = op_ref[pl.ds(t * _BMB, _BMB), :] + ctb

        @pl.when(fst == 0)
        def _():
            out_ref[...] = out_ref[...] + ctb

    @pl.when(rg_ref[i] == 0)
    def _():
        emit(adj_ref[...])

    @pl.when(rg_ref[i] == 1)
    def _():
        colid = jax.lax.broadcasted_iota(jnp.int32, adj_ref.shape, 1)
        emit(jnp.where(colid < _RAG, adj_ref[...], 0.0))

    @pl.when(lst_ref[i] == 1)
    def _():
        o = out_ref[...] + b2_ref[...]
        mx = jnp.max(o, axis=1, keepdims=True)
        shifted = o - mx
        lse = jnp.log(jnp.sum(jnp.exp(shifted), axis=1, keepdims=True))
        out_ref[...] = shifted - lse


def _pass_b_schedule(n):
    nb = n // _BMB
    ncb = -(-n // _CB)
    tb, cb, mz, fst, lst, rg = [], [], [], [], [], []
    for t in range(nb):
        end = (t + 1) * _BMB
        kend = -(-end // _CB)
        for c in range(kend):
            tb.append(t)
            cb.append(c)
            mz.append(_CB if c < kend - 1 else end - (kend - 1) * _CB)
            fst.append(1 if c == 0 else 0)
            lst.append(1 if c == kend - 1 else 0)
            rg.append(1 if c == ncb - 1 else 0)
    arrs = (tb, cb, mz, fst, lst, rg)
    return tuple(jnp.asarray(np.array(a, dtype=np.int32)) for a in arrs)


@jax.jit
def kernel(x, adj, W1, b1, W2, b2):
    n, nfeat = x.shape
    nhid = W1.shape[1]
    nclass = W2.shape[1]
    b1r = b1.reshape(1, nhid)
    b2r = b2.reshape(1, nclass)
    nb = n // _BM
    ncb = -(-n // _CB)

    h2, opart = pl.pallas_call(
        _pass_a_kernel,
        grid=(nb,),
        in_specs=[
            pl.BlockSpec(memory_space=pl.ANY),
            pl.BlockSpec((_BM, n), lambda i: (nb - 1 - i, 0)),
            pl.BlockSpec((nfeat, nhid), lambda i: (0, 0)),
            pl.BlockSpec((1, nhid), lambda i: (0, 0)),
            pl.BlockSpec((nhid, nclass), lambda i: (0, 0)),
        ],
        out_specs=[
            pl.BlockSpec((_BM, nclass), lambda i: (nb - 1 - i, 0)),
            pl.BlockSpec((_BM, nclass), lambda i: (nb - 1 - i, 0)),
        ],
        out_shape=[
            jax.ShapeDtypeStruct((n, nclass), jnp.float32),
            jax.ShapeDtypeStruct((n, nclass), jnp.float32),
        ],
        scratch_shapes=[
            pltpu.VMEM((n, nhid + nclass), jnp.float32),
            pltpu.VMEM((_BMB, nclass), jnp.float32),
            pltpu.VMEM((n, nfeat), jnp.float32),
            pltpu.SemaphoreType.DMA,
        ],
        compiler_params=pltpu.CompilerParams(
            dimension_semantics=("arbitrary",),
            vmem_limit_bytes=58 * 1024 * 1024),
    )(x, adj, W1, b1r, W2)

    h2p = jnp.zeros((ncb * _CB, nclass), jnp.float32).at[:n].set(h2)

    tb, cb, mz, fst, lst, rg = _pass_b_schedule(n)
    nsteps = tb.shape[0]

    grid_spec = pltpu.PrefetchScalarGridSpec(
        num_scalar_prefetch=6,
        grid=(nsteps,),
        in_specs=[
            pl.BlockSpec((_BMB, _CB),
                         lambda i, tb, cb, *_: (tb[i], cb[i])),
            pl.BlockSpec((ncb * _CB, nclass), lambda i, *_: (0, 0)),
            pl.BlockSpec((n, nclass), lambda i, *_: (0, 0)),
            pl.BlockSpec((1, nclass), lambda i, *_: (0, 0)),
        ],
        out_specs=pl.BlockSpec((_BMB, nclass),
                               lambda i, tb, *_: (tb[i], 0)),
    )

    return pl.pallas_call(
        _pass_b_kernel,
        grid_spec=grid_spec,
        out_shape=jax.ShapeDtypeStruct((n, nclass), jnp.float32),
        compiler_params=pltpu.CompilerParams(
            dimension_semantics=("arbitrary",),
            vmem_limit_bytes=58 * 1024 * 1024),
    )(tb, cb, mz, fst, lst, rg, adj, h2p, opart, b2r)


# speedup vs baseline: 1.6020x; 1.0140x over previous
import jax
import jax.numpy as jnp
import numpy as np
from jax.experimental import pallas as pl
from jax.experimental.pallas import tpu as pltpu

_BM = 400
_GB = 5
_BMB = _BM * _GB
_CB = 2048


def _pass_a_kernel(x_ref, adj_ref, w1_ref, b1_ref, w2_ref,
                   h2_out, op_out, sh_ref, hstage, x_s, x_sem):
    i = pl.program_id(0)
    nb = pl.num_programs(0)
    nhid = w1_ref.shape[1]
    nclass = w2_ref.shape[1]

    @pl.when(i == 0)
    def _():
        xcp = pltpu.make_async_copy(x_ref, x_s, x_sem)
        xcp.start()
        sh_ref[...] = jnp.zeros_like(sh_ref)
        xcp.wait()
        sh_ref[:, :nhid] = jnp.dot(x_s[...], w1_ref[...],
                                   preferred_element_type=jnp.float32)

    t = nb - 1 - i

    @pl.when(((t + 1) % _GB == 0) & (i > 0))
    def _():
        sh_ref[pl.ds((t + 1) * _BM, _BMB), nhid:nhid + nclass] = (
            hstage[...])

    a = adj_ref[...]
    big = jnp.dot(a, sh_ref[...], preferred_element_type=jnp.float32)
    hb = jnp.maximum(big[:, :nhid] + b1_ref[...], 0.0)
    h2blk = jnp.dot(hb, w2_ref[...], preferred_element_type=jnp.float32)
    hstage[pl.ds((t % _GB) * _BM, _BM), :] = h2blk
    h2_out[...] = h2blk
    op_out[...] = big[:, nhid:nhid + nclass]


_RAG = 10000 - 4 * _CB


def _pass_b_kernel(tb_ref, cb_ref, mz_ref, fst_ref, lst_ref, rg_ref,
                   adj_ref, h2_ref, op_ref, b2_ref, out_ref):
    i = pl.program_id(0)
    mz = mz_ref[i]
    c = cb_ref[i]
    t = tb_ref[i]
    fst = fst_ref[i]

    h2s = h2_ref[pl.ds(c * _CB, _CB), :]
    rowid = jax.lax.broadcasted_iota(jnp.int32, h2s.shape, 0)
    h2m = jnp.where(rowid < mz, h2s, 0.0)

    def emit(a_op):
        ctb = jnp.dot(a_op, h2m, preferred_element_type=jnp.float32)

        @pl.when(fst == 1)
        def _():
            out_ref[...] = op_ref[pl.ds(t * _BMB, _BMB), :] + ctb

        @pl.when(fst == 0)
        def _():
            out_ref[...] = out_ref[...] + ctb

    @pl.when(rg_ref[i] == 0)
    def _():
        emit(adj_ref[...])

    @pl.when(rg_ref[i] == 1)
    def _():
        colid = jax.lax.broadcasted_iota(jnp.int32, adj_ref.shape, 1)
        emit(jnp.where(colid < _RAG, adj_ref[...], 0.0))

    @pl.when(lst_ref[i] == 1)
    def _():
        o = out_ref[...] + b2_ref[...]
        mx = jnp.max(o, axis=1, keepdims=True)
        shifted = o - mx
        lse = jnp.log(jnp.sum(jnp.exp(shifted), axis=1, keepdims=True))
        out_ref[...] = shifted - lse


def _pass_b_schedule(n):
    nb = n // _BMB
    ncb = -(-n // _CB)
    tb, cb, mz, fst, lst, rg = [], [], [], [], [], []
    for t in range(nb):
        end = (t + 1) * _BMB
        kend = -(-end // _CB)
        for c in range(kend):
            tb.append(t)
            cb.append(c)
            mz.append(_CB if c < kend - 1 else end - (kend - 1) * _CB)
            fst.append(1 if c == 0 else 0)
            lst.append(1 if c == kend - 1 else 0)
            rg.append(1 if c == ncb - 1 else 0)
    arrs = (tb, cb, mz, fst, lst, rg)
    return tuple(jnp.asarray(np.array(a, dtype=np.int32)) for a in arrs)


@jax.jit
def kernel(x, adj, W1, b1, W2, b2):
    n, nfeat = x.shape
    nhid = W1.shape[1]
    nclass = W2.shape[1]
    b1r = b1.reshape(1, nhid)
    b2r = b2.reshape(1, nclass)
    nb = n // _BM
    ncb = -(-n // _CB)

    h2, opart = pl.pallas_call(
        _pass_a_kernel,
        grid=(nb,),
        in_specs=[
            pl.BlockSpec(memory_space=pl.ANY),
            pl.BlockSpec((_BM, n), lambda i: (nb - 1 - i, 0)),
            pl.BlockSpec((nfeat, nhid), lambda i: (0, 0)),
            pl.BlockSpec((1, nhid), lambda i: (0, 0)),
            pl.BlockSpec((nhid, nclass), lambda i: (0, 0)),
        ],
        out_specs=[
            pl.BlockSpec((_BM, nclass), lambda i: (nb - 1 - i, 0)),
            pl.BlockSpec((_BM, nclass), lambda i: (nb - 1 - i, 0)),
        ],
        out_shape=[
            jax.ShapeDtypeStruct((ncb * _CB, nclass), jnp.float32),
            jax.ShapeDtypeStruct((n, nclass), jnp.float32),
        ],
        scratch_shapes=[
            pltpu.VMEM((n, nhid + nclass), jnp.float32),
            pltpu.VMEM((_BMB, nclass), jnp.float32),
            pltpu.VMEM((n, nfeat), jnp.float32),
            pltpu.SemaphoreType.DMA,
        ],
        compiler_params=pltpu.CompilerParams(
            dimension_semantics=("arbitrary",),
            vmem_limit_bytes=58 * 1024 * 1024),
    )(x, adj, W1, b1r, W2)

    tb, cb, mz, fst, lst, rg = _pass_b_schedule(n)
    nsteps = tb.shape[0]

    grid_spec = pltpu.PrefetchScalarGridSpec(
        num_scalar_prefetch=6,
        grid=(nsteps,),
        in_specs=[
            pl.BlockSpec((_BMB, _CB),
                         lambda i, tb, cb, *_: (tb[i], cb[i])),
            pl.BlockSpec((ncb * _CB, nclass), lambda i, *_: (0, 0)),
            pl.BlockSpec((n, nclass), lambda i, *_: (0, 0)),
            pl.BlockSpec((1, nclass), lambda i, *_: (0, 0)),
        ],
        out_specs=pl.BlockSpec((_BMB, nclass),
                               lambda i, tb, *_: (tb[i], 0)),
    )

    return pl.pallas_call(
        _pass_b_kernel,
        grid_spec=grid_spec,
        out_shape=jax.ShapeDtypeStruct((n, nclass), jnp.float32),
        compiler_params=pltpu.CompilerParams(
            dimension_semantics=("arbitrary",),
            vmem_limit_bytes=58 * 1024 * 1024),
    )(tb, cb, mz, fst, lst, rg, adj, h2, opart, b2r)
